# baseline (speedup 1.0000x reference)
import functools

import jax
import jax.numpy as jnp
from jax import lax
from jax.experimental import pallas as pl
from jax.experimental.pallas import tpu as pltpu
from jax.experimental.pallas import tpu_sc as plsc

_NUM_CORES = 2
_NUM_SUBCORES = 16
_NUM_WORKERS = _NUM_CORES * _NUM_SUBCORES


def _gather_call(n_total, V, D, chunk):
  n_per_w = n_total // _NUM_WORKERS
  nchunks = n_per_w // chunk
  mesh = plsc.VectorSubcoreMesh(core_axis_name="c", subcore_axis_name="s")

  @functools.partial(
      pl.kernel,
      out_type=jax.ShapeDtypeStruct((n_total, D), jnp.float32),
      mesh=mesh,
      scratch_types=[
          pltpu.VMEM((n_per_w,), jnp.int32),
          pltpu.VMEM((chunk, D), jnp.float32),
          pltpu.SemaphoreType.DMA,
      ],
      compiler_params=pltpu.CompilerParams(use_tc_tiling_on_sc=False),
  )
  def body(idx_hbm, table_hbm, out_hbm, idx_v, rows_v, g_sem):
    wid = lax.axis_index("s") * _NUM_CORES + lax.axis_index("c")
    base = wid * n_per_w
    pltpu.sync_copy(idx_hbm.at[pl.ds(base, n_per_w)], idx_v)

    def step(c, carry):
      off = c * chunk
      pltpu.async_copy(
          table_hbm.at[idx_v.at[pl.ds(off, chunk)]], rows_v, g_sem
      ).wait()
      pltpu.sync_copy(rows_v, out_hbm.at[pl.ds(base + off, chunk)])
      return carry

    lax.fori_loop(0, nchunks, step, 0, unroll=False)

  return body


def kernel(context, table):
  B, T = context.shape
  V, D = table.shape
  n_total = B * T
  idx = context.reshape(n_total).astype(jnp.int32)
  out = _gather_call(n_total, V, D, chunk=40)(idx, table)
  return out.reshape(B, T, D)


# speedup vs baseline: 1.0441x; 1.0441x over previous
import functools

import jax
import jax.numpy as jnp
from jax import lax
from jax.experimental import pallas as pl
from jax.experimental.pallas import tpu as pltpu
from jax.experimental.pallas import tpu_sc as plsc

_NUM_CORES = 2
_NUM_SUBCORES = 16
_NUM_WORKERS = _NUM_CORES * _NUM_SUBCORES


def _gather_call(n_total, V, D, chunk):
  n_per_w = n_total // _NUM_WORKERS
  nchunks = n_per_w // chunk
  ngroups = nchunks // 2
  mesh = plsc.VectorSubcoreMesh(core_axis_name="c", subcore_axis_name="s")

  @functools.partial(
      pl.kernel,
      out_type=jax.ShapeDtypeStruct((n_total, D), jnp.float32),
      mesh=mesh,
      scratch_types=[
          pltpu.VMEM((n_per_w,), jnp.int32),
          pltpu.VMEM((chunk, D), jnp.float32),
          pltpu.VMEM((chunk, D), jnp.float32),
          pltpu.SemaphoreType.DMA,
          pltpu.SemaphoreType.DMA,
          pltpu.SemaphoreType.DMA,
          pltpu.SemaphoreType.DMA,
      ],
      compiler_params=pltpu.CompilerParams(use_tc_tiling_on_sc=False),
  )
  def body(idx_hbm, table_hbm, out_hbm, idx_v, rows0, rows1, g0, g1, w0, w1):
    wid = lax.axis_index("s") * _NUM_CORES + lax.axis_index("c")
    base = wid * n_per_w
    pltpu.sync_copy(idx_hbm.at[pl.ds(base, n_per_w)], idx_v)

    def fire_gather(c, buf, sem):
      pltpu.async_copy(table_hbm.at[idx_v.at[pl.ds(c * chunk, chunk)]], buf, sem)

    def fire_wb(c, buf, sem):
      pltpu.async_copy(buf, out_hbm.at[pl.ds(base + c * chunk, chunk)], sem)

    def wait_gather(buf, sem):
      pltpu.make_async_copy(out_hbm.at[pl.ds(base, chunk)], buf, sem).wait()

    def wait_wb(buf, sem):
      pltpu.make_async_copy(buf, out_hbm.at[pl.ds(base, chunk)], sem).wait()

    fire_gather(0, rows0, g0)
    fire_gather(1, rows1, g1)
    wait_gather(rows0, g0)
    fire_wb(0, rows0, w0)

    def step(k, carry):
      c0 = 2 * k
      wait_wb(rows0, w0)
      fire_gather(c0, rows0, g0)
      wait_gather(rows1, g1)
      fire_wb(c0 - 1, rows1, w1)
      wait_wb(rows1, w1)
      fire_gather(c0 + 1, rows1, g1)
      wait_gather(rows0, g0)
      fire_wb(c0, rows0, w0)
      return carry

    lax.fori_loop(1, ngroups, step, 0, unroll=False)

    wait_gather(rows1, g1)
    fire_wb(nchunks - 1, rows1, w1)
    wait_wb(rows0, w0)
    wait_wb(rows1, w1)

  return body


def kernel(context, table):
  B, T = context.shape
  V, D = table.shape
  n_total = B * T
  idx = context.reshape(n_total).astype(jnp.int32)
  out = _gather_call(n_total, V, D, chunk=40)(idx, table)
  return out.reshape(B, T, D)


# speedup vs baseline: 1.8039x; 1.7278x over previous
import functools

import jax
import jax.numpy as jnp
from jax import lax
from jax.experimental import pallas as pl
from jax.experimental.pallas import tpu as pltpu
from jax.experimental.pallas import tpu_sc as plsc

_NUM_CORES = 2
_NUM_SUBCORES = 16
_NUM_WORKERS = _NUM_CORES * _NUM_SUBCORES


def _gather_call(n_total, D, Dp, chunk):
  n_per_w = n_total // _NUM_WORKERS
  nchunks = n_per_w // chunk
  ngroups = nchunks // 2
  mesh = plsc.VectorSubcoreMesh(core_axis_name="c", subcore_axis_name="s")

  @functools.partial(
      pl.kernel,
      out_type=jax.ShapeDtypeStruct((n_total, Dp), jnp.float32),
      mesh=mesh,
      scratch_types=[
          pltpu.VMEM((n_per_w,), jnp.int32),
          pltpu.VMEM((chunk, Dp), jnp.float32),
          pltpu.VMEM((chunk, Dp), jnp.float32),
          pltpu.SemaphoreType.DMA,
          pltpu.SemaphoreType.DMA,
          pltpu.SemaphoreType.DMA,
          pltpu.SemaphoreType.DMA,
      ],
  )
  def body(idx_hbm, table_hbm, out_hbm, idx_v, rows0, rows1, g0, g1, w0, w1):
    wid = lax.axis_index("s") * _NUM_CORES + lax.axis_index("c")
    base = wid * n_per_w
    pltpu.sync_copy(idx_hbm.at[pl.ds(base, n_per_w)], idx_v)

    def fire_gather(c, buf, sem):
      pltpu.async_copy(table_hbm.at[idx_v.at[pl.ds(c * chunk, chunk)]], buf, sem)

    def fire_wb(c, buf, sem):
      pltpu.async_copy(buf, out_hbm.at[pl.ds(base + c * chunk, chunk)], sem)

    def wait_gather(buf, sem):
      pltpu.make_async_copy(table_hbm.at[pl.ds(0, chunk)], buf, sem).wait()

    def wait_wb(buf, sem):
      pltpu.make_async_copy(buf, out_hbm.at[pl.ds(base, chunk)], sem).wait()

    fire_gather(0, rows0, g0)
    fire_gather(1, rows1, g1)
    wait_gather(rows0, g0)
    fire_wb(0, rows0, w0)

    def step(k, carry):
      c0 = 2 * k
      wait_wb(rows0, w0)
      fire_gather(c0, rows0, g0)
      wait_gather(rows1, g1)
      fire_wb(c0 - 1, rows1, w1)
      wait_wb(rows1, w1)
      fire_gather(c0 + 1, rows1, g1)
      wait_gather(rows0, g0)
      fire_wb(c0, rows0, w0)
      return carry

    lax.fori_loop(1, ngroups, step, 0, unroll=False)

    wait_gather(rows1, g1)
    fire_wb(nchunks - 1, rows1, w1)
    wait_wb(rows0, w0)
    wait_wb(rows1, w1)

  return body


def kernel(context, table):
  B, T = context.shape
  V, D = table.shape
  n_total = B * T
  pad = (-D) % 128
  Dp = D + pad
  idx = context.reshape(n_total).astype(jnp.int32)
  table_p = jnp.pad(table, ((0, 0), (0, pad)))
  out = _gather_call(n_total, D, Dp, chunk=40)(idx, table_p)
  return out[:, :D].reshape(B, T, D)


# speedup vs baseline: 1.8045x; 1.0004x over previous
import functools

import jax
import jax.numpy as jnp
from jax import lax
from jax.experimental import pallas as pl
from jax.experimental.pallas import tpu as pltpu
from jax.experimental.pallas import tpu_sc as plsc

_NUM_CORES = 2
_NUM_SUBCORES = 16
_NUM_WORKERS = _NUM_CORES * _NUM_SUBCORES


def _gather_call(n_total, D, Dp, chunk):
  n_per_w = n_total // _NUM_WORKERS
  nchunks = n_per_w // chunk
  ngroups = nchunks // 2
  mesh = plsc.VectorSubcoreMesh(core_axis_name="c", subcore_axis_name="s")

  @functools.partial(
      pl.kernel,
      out_type=jax.ShapeDtypeStruct((n_total, D), jnp.float32),
      mesh=mesh,
      scratch_types=[
          pltpu.VMEM((n_per_w,), jnp.int32),
          pltpu.VMEM((chunk, Dp), jnp.float32),
          pltpu.VMEM((chunk, Dp), jnp.float32),
          pltpu.SemaphoreType.DMA,
          pltpu.SemaphoreType.DMA,
          pltpu.SemaphoreType.DMA,
          pltpu.SemaphoreType.DMA,
      ],
  )
  def body(idx_hbm, table_hbm, out_hbm, idx_v, rows0, rows1, g0, g1, w0, w1):
    wid = lax.axis_index("s") * _NUM_CORES + lax.axis_index("c")
    base = wid * n_per_w
    pltpu.sync_copy(idx_hbm.at[pl.ds(base, n_per_w)], idx_v)

    def fire_gather(c, buf, sem):
      pltpu.async_copy(table_hbm.at[idx_v.at[pl.ds(c * chunk, chunk)]], buf, sem)

    def fire_wb(c, buf, sem):
      pltpu.async_copy(buf, out_hbm.at[pl.ds(base + c * chunk, chunk), pl.ds(0, Dp)], sem)

    def wait_gather(buf, sem):
      pltpu.make_async_copy(table_hbm.at[pl.ds(0, chunk)], buf, sem).wait()

    def wait_wb(buf, sem):
      pltpu.make_async_copy(buf, out_hbm.at[pl.ds(base, chunk), pl.ds(0, Dp)], sem).wait()

    fire_gather(0, rows0, g0)
    fire_gather(1, rows1, g1)
    wait_gather(rows0, g0)
    fire_wb(0, rows0, w0)

    def step(k, carry):
      c0 = 2 * k
      wait_wb(rows0, w0)
      fire_gather(c0, rows0, g0)
      wait_gather(rows1, g1)
      fire_wb(c0 - 1, rows1, w1)
      wait_wb(rows1, w1)
      fire_gather(c0 + 1, rows1, g1)
      wait_gather(rows0, g0)
      fire_wb(c0, rows0, w0)
      return carry

    lax.fori_loop(1, ngroups, step, 0, unroll=False)

    wait_gather(rows1, g1)
    fire_wb(nchunks - 1, rows1, w1)
    wait_wb(rows0, w0)
    wait_wb(rows1, w1)

  return body


def kernel(context, table):
  B, T = context.shape
  V, D = table.shape
  n_total = B * T
  pad = (-D) % 128
  Dp = D + pad
  idx = context.reshape(n_total).astype(jnp.int32)
  table_p = jnp.pad(table, ((0, 0), (0, pad)))
  out = _gather_call(n_total, D, Dp, chunk=40)(idx, table_p)
  return out.reshape(B, T, D)
